# baseline (speedup 1.0000x reference)
import functools

import jax
import jax.numpy as jnp
from jax import lax
from jax.experimental import pallas as pl
from jax.experimental.pallas import tpu as pltpu
from jax.experimental.pallas import tpu_sc as plsc

_NUM_CLASSES = 4
_PROTO_MOMENTUM = 0.9
_CONF_THRESH = 0.8
_LAMBDA_INTRA = 1.0
_LAMBDA_INTER = 0.1
_MARGIN_M = 1.0

_NV = 1024
_ROWS = 5 * 66
_NW = 32
_TC_CHUNK = 32768
_SC_FRAC = 0.375


def _sc_body(feat_hbm, pred_hbm, label_hbm, islab_hbm, out_hbm,
             featbuf, predbuf, labelbuf, islabbuf, acc, *, B, K, S):
    wid = lax.axis_index("s") * 2 + lax.axis_index("c")
    span = S // _NW
    nchunks = span // _NV
    ngroups = _NV // 16

    lane = lax.iota(jnp.int32, 16)
    zeros16 = jnp.zeros((16,), jnp.float32)
    ones16 = jnp.ones((16,), jnp.float32)

    def _zero(i, _):
        acc[pl.ds(i * 16, 16)] = zeros16
        return 0
    lax.fori_loop(0, _ROWS, _zero, 0)

    pltpu.sync_copy(islab_hbm, islabbuf)

    for b in range(B):
        islab_v = islabbuf[b]
        for ch in range(nchunks):
            off = wid * span + ch * _NV
            pltpu.sync_copy(feat_hbm.at[b, :, pl.ds(off, _NV)], featbuf)
            pltpu.sync_copy(pred_hbm.at[b, :, pl.ds(off, _NV)], predbuf)
            pltpu.sync_copy(label_hbm.at[b, pl.ds(off, _NV)], labelbuf)

            def _group(g, _):
                g16 = g * 16
                conf = predbuf[0, pl.ds(g16, 16)]
                cls = jnp.zeros((16,), jnp.int32)
                for k in range(1, K):
                    pk = predbuf[k, pl.ds(g16, 16)]
                    better = pk > conf
                    conf = jnp.where(better, pk, conf)
                    cls = jnp.where(better, k, cls)
                lbl = labelbuf[pl.ds(g16, 16)]
                sel = (conf > _CONF_THRESH) & (cls > 0)
                sel = sel & ((cls == lbl) | (islab_v < 0.5))
                cls_eff = jnp.where(sel, cls, 0)
                idx = cls_eff * (66 * 16) + lane
                s2v = zeros16
                for c in range(64):
                    fv = featbuf[c, pl.ds(g16, 16)]
                    plsc.addupdate_scatter(acc, [idx + c * 16], fv)
                    s2v = s2v + fv * fv
                plsc.addupdate_scatter(acc, [idx + 64 * 16], ones16)
                plsc.addupdate_scatter(acc, [idx + 65 * 16], s2v)
                return 0
            lax.fori_loop(0, ngroups, _group, 0)

    pltpu.sync_copy(acc, out_hbm.at[wid])


def _tcs_body(pred_ref, label_ref, islab_ref, feat_ref,
              sums_ref, cnt_ref, s2_ref, *, kcls):
    b = pl.program_id(0)
    i = pl.program_id(1)
    first = (b == 0) & (i == 0)

    p = pred_ref[0]
    lbl = label_ref[0]
    islab = islab_ref[0, 0, 0]

    conf = p[0:1]
    cls = jnp.zeros_like(lbl)
    for k in range(1, kcls):
        pk = p[k:k + 1]
        better = pk > conf
        conf = jnp.where(better, pk, conf)
        cls = jnp.where(better, k, cls)

    mask = (conf > _CONF_THRESH) & (cls > 0)
    mask &= (cls == lbl) | (islab < 0.5)
    w = mask.astype(jnp.float32)

    kv = jax.lax.broadcasted_iota(jnp.int32, (_NUM_CLASSES, 1), 0) + 1
    w4 = jnp.where(cls == kv, w, 0.0)

    dot = functools.partial(jax.lax.dot_general,
                            precision=jax.lax.Precision.DEFAULT,
                            preferred_element_type=jnp.float32)
    f = feat_ref[0]
    dsums = dot(w4, f, (((1,), (1,)), ((), ())))
    s2 = jnp.sum(f * f, axis=0, keepdims=True)
    dcnt = jnp.sum(w4, axis=1, keepdims=True)
    ds2 = dot(w4, s2, (((1,), (1,)), ((), ())))

    @pl.when(first)
    def _init():
        sums_ref[...] = dsums
        cnt_ref[...] = dcnt
        s2_ref[...] = ds2

    @pl.when(jnp.logical_not(first))
    def _acc():
        sums_ref[...] += dsums
        cnt_ref[...] += dcnt
        s2_ref[...] += ds2


def _epi_body(p_ref, tsums_ref, tcnt_ref, ts2_ref, proto_ref, pinit_ref,
              out_ref, *, nw):
    x = p_ref[...]
    y = x[0]
    for w in range(1, nw):
        y = y + x[w]
    rowsum = jnp.sum(y, axis=1, keepdims=True)

    hi = functools.partial(jax.lax.dot_general,
                           precision=jax.lax.Precision.HIGHEST,
                           preferred_element_type=jnp.float32)
    rr = jax.lax.broadcasted_iota(jnp.int32, (_ROWS, 64), 0)
    cc = jax.lax.broadcasted_iota(jnp.int32, (_ROWS, 64), 1)
    chm = ((rr % 66) == cc).astype(jnp.float32)
    kk = jax.lax.broadcasted_iota(jnp.int32, (_NUM_CLASSES, _ROWS), 0)
    rr2 = jax.lax.broadcasted_iota(jnp.int32, (_NUM_CLASSES, _ROWS), 1)
    ks = ((rr2 // 66) == (kk + 1)).astype(jnp.float32)
    rr1 = jax.lax.broadcasted_iota(jnp.int32, (_ROWS, 1), 0)

    sums = hi(ks, rowsum * chm, (((1,), (0,)), ((), ()))) + tsums_ref[...]
    counts = (hi(ks, rowsum * ((rr1 % 66) == 64), (((1,), (0,)), ((), ())))
              + tcnt_ref[...])
    s2sum = (hi(ks, rowsum * ((rr1 % 66) == 65), (((1,), (0,)), ((), ())))
             + ts2_ref[...])

    proto = proto_ref[...]
    pinit = pinit_ref[...] > 0.5

    present = counts > 0.0
    new_proto = sums / jnp.maximum(counts, 1.0)
    updated = jnp.where(present,
                        jnp.where(pinit,
                                  _PROTO_MOMENTUM * proto
                                  + (1.0 - _PROTO_MOMENTUM) * new_proto,
                                  new_proto),
                        proto)
    ini = pinit | present

    un2 = jnp.sum(updated * updated, axis=1, keepdims=True)
    total = (jnp.sum(s2sum)
             - 2.0 * jnp.sum(sums * updated)
             + jnp.sum(counts * un2))
    vp = jnp.sum(counts)
    loss_intra = jnp.where(vp > 0.0, total / jnp.maximum(vp, 1.0), 0.0)

    gram = hi(updated, updated, (((1,), (1,)), ((), ())))
    d2m = un2 + jnp.reshape(un2, (1, _NUM_CLASSES)) - 2.0 * gram
    dist = jnp.sqrt(jnp.maximum(d2m, 0.0) + 1e-12)
    r = jax.lax.broadcasted_iota(jnp.int32, (_NUM_CLASSES, _NUM_CLASSES), 0)
    c = jax.lax.broadcasted_iota(jnp.int32, (_NUM_CLASSES, _NUM_CLASSES), 1)
    pair_valid = (ini & jnp.reshape(ini, (1, _NUM_CLASSES)) & (c > r))
    pen = jnp.maximum(_MARGIN_M - dist, 0.0) ** 2
    pvf = pair_valid.astype(jnp.float32)
    n_pairs = jnp.sum(pvf)
    loss_inter = jnp.where(n_pairs > 0.0,
                           jnp.sum(pen * pvf) / jnp.maximum(n_pairs, 1.0),
                           0.0)
    loss = _LAMBDA_INTRA * loss_intra + _LAMBDA_INTER * loss_inter
    out_ref[...] = jnp.reshape(loss, (1, 1))


def kernel(feat, pred, label, is_labelled, prototypes, prototype_initialized):
    B, C, H, W, D = feat.shape
    K = pred.shape[1]
    M = H * W * D

    gran = _NW * _NV
    S = (int(M * _SC_FRAC) // gran) * gran
    rest = M - S
    ctc = min(_TC_CHUNK, rest)
    while rest % ctc:
        ctc //= 2
    nc_tc = rest // ctc
    off_blocks = S // ctc

    feat3 = feat.reshape(B, C, M)
    pred3 = pred.reshape(B, K, M)
    label2 = label.reshape(B, M)
    label3 = label.reshape(B, 1, M)
    islab16 = jnp.broadcast_to(
        is_labelled.astype(jnp.float32).reshape(B, 1), (B, 16))
    islab = is_labelled.astype(jnp.float32).reshape(B, 1, 1)
    pinit = prototype_initialized.astype(jnp.float32).reshape(_NUM_CLASSES, 1)

    tsums, tcnt, ts2 = pl.pallas_call(
        functools.partial(_tcs_body, kcls=K),
        grid=(B, nc_tc),
        in_specs=[
            pl.BlockSpec((1, K, ctc),
                         lambda b, i: (b, 0, i + off_blocks)),
            pl.BlockSpec((1, 1, ctc),
                         lambda b, i: (b, 0, i + off_blocks)),
            pl.BlockSpec((1, 1, 1), lambda b, i: (b, 0, 0)),
            pl.BlockSpec((1, C, ctc),
                         lambda b, i: (b, 0, i + off_blocks)),
        ],
        out_specs=[
            pl.BlockSpec((_NUM_CLASSES, C), lambda b, i: (0, 0)),
            pl.BlockSpec((_NUM_CLASSES, 1), lambda b, i: (0, 0)),
            pl.BlockSpec((_NUM_CLASSES, 1), lambda b, i: (0, 0)),
        ],
        out_shape=[
            jax.ShapeDtypeStruct((_NUM_CLASSES, C), jnp.float32),
            jax.ShapeDtypeStruct((_NUM_CLASSES, 1), jnp.float32),
            jax.ShapeDtypeStruct((_NUM_CLASSES, 1), jnp.float32),
        ],
    )(pred3, label3, islab, feat3)

    mesh = plsc.VectorSubcoreMesh(core_axis_name="c", subcore_axis_name="s")
    sck = functools.partial(
        pl.kernel,
        mesh=mesh,
        compiler_params=pltpu.CompilerParams(needs_layout_passes=False, has_side_effects=False),
        out_type=jax.ShapeDtypeStruct((_NW, _ROWS * 16), jnp.float32),
        scratch_types=[
            pltpu.VMEM((C, _NV), jnp.float32),
            pltpu.VMEM((K, _NV), jnp.float32),
            pltpu.VMEM((_NV,), jnp.int32),
            pltpu.VMEM((B, 16), jnp.float32),
            pltpu.VMEM((_ROWS * 16,), jnp.float32),
        ],
    )(functools.partial(_sc_body, B=B, K=K, S=S))
    partials = sck(feat3, pred3, label2, islab16)

    out = pl.pallas_call(
        functools.partial(_epi_body, nw=_NW),
        in_specs=[
            pl.BlockSpec((_NW, _ROWS, 16), lambda: (0, 0, 0)),
            pl.BlockSpec((_NUM_CLASSES, C), lambda: (0, 0)),
            pl.BlockSpec((_NUM_CLASSES, 1), lambda: (0, 0)),
            pl.BlockSpec((_NUM_CLASSES, 1), lambda: (0, 0)),
            pl.BlockSpec((_NUM_CLASSES, C), lambda: (0, 0)),
            pl.BlockSpec((_NUM_CLASSES, 1), lambda: (0, 0)),
        ],
        out_specs=pl.BlockSpec((1, 1), lambda: (0, 0)),
        out_shape=jax.ShapeDtypeStruct((1, 1), jnp.float32),
    )(partials.reshape(_NW, _ROWS, 16), tsums, tcnt, ts2,
      prototypes, pinit)
    return out.reshape(())


# speedup vs baseline: 1.1864x; 1.1864x over previous
import functools

import jax
import jax.numpy as jnp
from jax import lax
from jax.experimental import pallas as pl
from jax.experimental.pallas import tpu as pltpu
from jax.experimental.pallas import tpu_sc as plsc

_NUM_CLASSES = 4
_PROTO_MOMENTUM = 0.9
_CONF_THRESH = 0.8
_LAMBDA_INTRA = 1.0
_LAMBDA_INTER = 0.1
_MARGIN_M = 1.0

_NV = 1024
_ROWS = 5 * 66
_NW = 32
_TC_CHUNK = 32768
_SC_FRAC = 0.125


def _sc_body(feat_hbm, pred_hbm, label_hbm, islab_hbm, out_hbm,
             featbuf, predbuf, labelbuf, islabbuf, acc, *, B, K, S):
    wid = lax.axis_index("s") * 2 + lax.axis_index("c")
    span = S // _NW
    nchunks = span // _NV
    ngroups = _NV // 16

    lane = lax.iota(jnp.int32, 16)
    zeros16 = jnp.zeros((16,), jnp.float32)
    ones16 = jnp.ones((16,), jnp.float32)

    def _zero(i, _):
        acc[pl.ds(i * 16, 16)] = zeros16
        return 0
    lax.fori_loop(0, _ROWS, _zero, 0)

    pltpu.sync_copy(islab_hbm, islabbuf)

    for b in range(B):
        islab_v = islabbuf[b]
        for ch in range(nchunks):
            off = wid * span + ch * _NV
            pltpu.sync_copy(feat_hbm.at[b, :, pl.ds(off, _NV)], featbuf)
            pltpu.sync_copy(pred_hbm.at[b, :, pl.ds(off, _NV)], predbuf)
            pltpu.sync_copy(label_hbm.at[b, pl.ds(off, _NV)], labelbuf)

            def _group(g, _):
                g16 = g * 16
                conf = predbuf[0, pl.ds(g16, 16)]
                cls = jnp.zeros((16,), jnp.int32)
                for k in range(1, K):
                    pk = predbuf[k, pl.ds(g16, 16)]
                    better = pk > conf
                    conf = jnp.where(better, pk, conf)
                    cls = jnp.where(better, k, cls)
                lbl = labelbuf[pl.ds(g16, 16)]
                sel = (conf > _CONF_THRESH) & (cls > 0)
                sel = sel & ((cls == lbl) | (islab_v < 0.5))
                cls_eff = jnp.where(sel, cls, 0)
                idx = cls_eff * (66 * 16) + lane
                s2v = zeros16
                for c in range(64):
                    fv = featbuf[c, pl.ds(g16, 16)]
                    plsc.addupdate_scatter(acc, [idx + c * 16], fv)
                    s2v = s2v + fv * fv
                plsc.addupdate_scatter(acc, [idx + 64 * 16], ones16)
                plsc.addupdate_scatter(acc, [idx + 65 * 16], s2v)
                return 0
            lax.fori_loop(0, ngroups, _group, 0)

    pltpu.sync_copy(acc, out_hbm.at[wid])


def _tcs_body(pred_ref, label_ref, islab_ref, feat_ref,
              sums_ref, cnt_ref, s2_ref, *, kcls):
    b = pl.program_id(0)
    i = pl.program_id(1)
    first = (b == 0) & (i == 0)

    p = pred_ref[0]
    lbl = label_ref[0]
    islab = islab_ref[0, 0, 0]

    conf = p[0:1]
    cls = jnp.zeros_like(lbl)
    for k in range(1, kcls):
        pk = p[k:k + 1]
        better = pk > conf
        conf = jnp.where(better, pk, conf)
        cls = jnp.where(better, k, cls)

    mask = (conf > _CONF_THRESH) & (cls > 0)
    mask &= (cls == lbl) | (islab < 0.5)
    w = mask.astype(jnp.float32)

    kv = jax.lax.broadcasted_iota(jnp.int32, (_NUM_CLASSES, 1), 0) + 1
    w4 = jnp.where(cls == kv, w, 0.0)

    dot = functools.partial(jax.lax.dot_general,
                            precision=jax.lax.Precision.DEFAULT,
                            preferred_element_type=jnp.float32)
    f = feat_ref[0]
    dsums = dot(w4, f, (((1,), (1,)), ((), ())))
    s2 = jnp.sum(f * f, axis=0, keepdims=True)
    dcnt = jnp.sum(w4, axis=1, keepdims=True)
    ds2 = dot(w4, s2, (((1,), (1,)), ((), ())))

    @pl.when(first)
    def _init():
        sums_ref[...] = dsums
        cnt_ref[...] = dcnt
        s2_ref[...] = ds2

    @pl.when(jnp.logical_not(first))
    def _acc():
        sums_ref[...] += dsums
        cnt_ref[...] += dcnt
        s2_ref[...] += ds2


def _epi_body(p_ref, tsums_ref, tcnt_ref, ts2_ref, proto_ref, pinit_ref,
              out_ref, *, nw):
    x = p_ref[...]
    y = x[0]
    for w in range(1, nw):
        y = y + x[w]
    rowsum = jnp.sum(y, axis=1, keepdims=True)

    hi = functools.partial(jax.lax.dot_general,
                           precision=jax.lax.Precision.HIGHEST,
                           preferred_element_type=jnp.float32)
    rr = jax.lax.broadcasted_iota(jnp.int32, (_ROWS, 64), 0)
    cc = jax.lax.broadcasted_iota(jnp.int32, (_ROWS, 64), 1)
    chm = ((rr % 66) == cc).astype(jnp.float32)
    kk = jax.lax.broadcasted_iota(jnp.int32, (_NUM_CLASSES, _ROWS), 0)
    rr2 = jax.lax.broadcasted_iota(jnp.int32, (_NUM_CLASSES, _ROWS), 1)
    ks = ((rr2 // 66) == (kk + 1)).astype(jnp.float32)
    rr1 = jax.lax.broadcasted_iota(jnp.int32, (_ROWS, 1), 0)

    sums = hi(ks, rowsum * chm, (((1,), (0,)), ((), ()))) + tsums_ref[...]
    counts = (hi(ks, rowsum * ((rr1 % 66) == 64), (((1,), (0,)), ((), ())))
              + tcnt_ref[...])
    s2sum = (hi(ks, rowsum * ((rr1 % 66) == 65), (((1,), (0,)), ((), ())))
             + ts2_ref[...])

    proto = proto_ref[...]
    pinit = pinit_ref[...] > 0.5

    present = counts > 0.0
    new_proto = sums / jnp.maximum(counts, 1.0)
    updated = jnp.where(present,
                        jnp.where(pinit,
                                  _PROTO_MOMENTUM * proto
                                  + (1.0 - _PROTO_MOMENTUM) * new_proto,
                                  new_proto),
                        proto)
    ini = pinit | present

    un2 = jnp.sum(updated * updated, axis=1, keepdims=True)
    total = (jnp.sum(s2sum)
             - 2.0 * jnp.sum(sums * updated)
             + jnp.sum(counts * un2))
    vp = jnp.sum(counts)
    loss_intra = jnp.where(vp > 0.0, total / jnp.maximum(vp, 1.0), 0.0)

    gram = hi(updated, updated, (((1,), (1,)), ((), ())))
    d2m = un2 + jnp.reshape(un2, (1, _NUM_CLASSES)) - 2.0 * gram
    dist = jnp.sqrt(jnp.maximum(d2m, 0.0) + 1e-12)
    r = jax.lax.broadcasted_iota(jnp.int32, (_NUM_CLASSES, _NUM_CLASSES), 0)
    c = jax.lax.broadcasted_iota(jnp.int32, (_NUM_CLASSES, _NUM_CLASSES), 1)
    pair_valid = (ini & jnp.reshape(ini, (1, _NUM_CLASSES)) & (c > r))
    pen = jnp.maximum(_MARGIN_M - dist, 0.0) ** 2
    pvf = pair_valid.astype(jnp.float32)
    n_pairs = jnp.sum(pvf)
    loss_inter = jnp.where(n_pairs > 0.0,
                           jnp.sum(pen * pvf) / jnp.maximum(n_pairs, 1.0),
                           0.0)
    loss = _LAMBDA_INTRA * loss_intra + _LAMBDA_INTER * loss_inter
    out_ref[...] = jnp.reshape(loss, (1, 1))


def kernel(feat, pred, label, is_labelled, prototypes, prototype_initialized):
    B, C, H, W, D = feat.shape
    K = pred.shape[1]
    M = H * W * D

    gran = _NW * _NV
    S = (int(M * _SC_FRAC) // gran) * gran
    rest = M - S
    ctc = min(_TC_CHUNK, rest)
    while rest % ctc:
        ctc //= 2
    nc_tc = rest // ctc
    off_blocks = S // ctc

    feat3 = feat.reshape(B, C, M)
    pred3 = pred.reshape(B, K, M)
    label2 = label.reshape(B, M)
    label3 = label.reshape(B, 1, M)
    islab16 = jnp.broadcast_to(
        is_labelled.astype(jnp.float32).reshape(B, 1), (B, 16))
    islab = is_labelled.astype(jnp.float32).reshape(B, 1, 1)
    pinit = prototype_initialized.astype(jnp.float32).reshape(_NUM_CLASSES, 1)

    tsums, tcnt, ts2 = pl.pallas_call(
        functools.partial(_tcs_body, kcls=K),
        grid=(B, nc_tc),
        in_specs=[
            pl.BlockSpec((1, K, ctc),
                         lambda b, i: (b, 0, i + off_blocks)),
            pl.BlockSpec((1, 1, ctc),
                         lambda b, i: (b, 0, i + off_blocks)),
            pl.BlockSpec((1, 1, 1), lambda b, i: (b, 0, 0)),
            pl.BlockSpec((1, C, ctc),
                         lambda b, i: (b, 0, i + off_blocks)),
        ],
        out_specs=[
            pl.BlockSpec((_NUM_CLASSES, C), lambda b, i: (0, 0)),
            pl.BlockSpec((_NUM_CLASSES, 1), lambda b, i: (0, 0)),
            pl.BlockSpec((_NUM_CLASSES, 1), lambda b, i: (0, 0)),
        ],
        out_shape=[
            jax.ShapeDtypeStruct((_NUM_CLASSES, C), jnp.float32),
            jax.ShapeDtypeStruct((_NUM_CLASSES, 1), jnp.float32),
            jax.ShapeDtypeStruct((_NUM_CLASSES, 1), jnp.float32),
        ],
    )(pred3, label3, islab, feat3)

    mesh = plsc.VectorSubcoreMesh(core_axis_name="c", subcore_axis_name="s")
    sck = functools.partial(
        pl.kernel,
        mesh=mesh,
        compiler_params=pltpu.CompilerParams(needs_layout_passes=False, has_side_effects=False),
        out_type=jax.ShapeDtypeStruct((_NW, _ROWS * 16), jnp.float32),
        scratch_types=[
            pltpu.VMEM((C, _NV), jnp.float32),
            pltpu.VMEM((K, _NV), jnp.float32),
            pltpu.VMEM((_NV,), jnp.int32),
            pltpu.VMEM((B, 16), jnp.float32),
            pltpu.VMEM((_ROWS * 16,), jnp.float32),
        ],
    )(functools.partial(_sc_body, B=B, K=K, S=S))
    partials = sck(feat3, pred3, label2, islab16)

    out = pl.pallas_call(
        functools.partial(_epi_body, nw=_NW),
        in_specs=[
            pl.BlockSpec((_NW, _ROWS, 16), lambda: (0, 0, 0)),
            pl.BlockSpec((_NUM_CLASSES, C), lambda: (0, 0)),
            pl.BlockSpec((_NUM_CLASSES, 1), lambda: (0, 0)),
            pl.BlockSpec((_NUM_CLASSES, 1), lambda: (0, 0)),
            pl.BlockSpec((_NUM_CLASSES, C), lambda: (0, 0)),
            pl.BlockSpec((_NUM_CLASSES, 1), lambda: (0, 0)),
        ],
        out_specs=pl.BlockSpec((1, 1), lambda: (0, 0)),
        out_shape=jax.ShapeDtypeStruct((1, 1), jnp.float32),
    )(partials.reshape(_NW, _ROWS, 16), tsums, tcnt, ts2,
      prototypes, pinit)
    return out.reshape(())
